# baseline (speedup 1.0000x reference)
import functools
import math

import jax
import jax.numpy as jnp
from jax import lax
from jax.experimental import pallas as pl
from jax.experimental.pallas import tpu as pltpu
from jax.experimental.pallas import tpu_sc as plsc

_INFO = plsc.get_sparse_core_info()
_NC = _INFO.num_cores
_NS = _INFO.num_subcores
_L = _INFO.num_lanes
_NW = _NC * _NS

_CH = 128


@functools.lru_cache(maxsize=None)
def _make_gather(B, D, num_objs):
    b_per_w = B // _NW
    n_ch = b_per_w // _CH
    mesh = plsc.VectorSubcoreMesh(core_axis_name="c", subcore_axis_name="s")

    @functools.partial(
        pl.kernel,
        mesh=mesh,
        out_type=jax.ShapeDtypeStruct((D, B), jnp.float32),
        compiler_params=pltpu.CompilerParams(needs_layout_passes=False),
        scratch_types=[
            pltpu.VMEM((b_per_w,), jnp.int32),
            pltpu.VMEM((b_per_w,), jnp.int32),
            pltpu.VMEM((n_ch, _CH), jnp.int32),
            pltpu.VMEM((b_per_w,), jnp.int32),
            pltpu.VMEM((b_per_w, 2 * D), jnp.float32),
            pltpu.VMEM((D, b_per_w), jnp.float32),
            pltpu.SemaphoreType.DMA,
        ],
    )
    def gather_kernel(l0_hbm, l1_hbm, tab_hbm, out_hbm,
                      l0_v, l1_v, arow_v, hoff_v, rows_v, outt_v, sem):
        wid = lax.axis_index("s") * _NC + lax.axis_index("c")
        base = wid * b_per_w
        pltpu.sync_copy(l0_hbm.at[pl.ds(base, b_per_w)], l0_v)
        pltpu.sync_copy(l1_hbm.at[pl.ds(base, b_per_w)], l1_v)
        for j in range(n_ch):
            for i in range(_CH // _L):
                off = j * _CH + i * _L
                a = l0_v[pl.ds(off, _L)]
                b = l1_v[pl.ds(off, _L)]
                idx = a * num_objs + b
                arow_v[j, pl.ds(i * _L, _L)] = lax.shift_right_logical(idx, 1)
                hoff_v[pl.ds(off, _L)] = (idx & 1) * D
        copies = [
            pltpu.async_copy(tab_hbm.at[arow_v.at[j]],
                             rows_v.at[pl.ds(j * _CH, _CH)], sem)
            for j in range(n_ch)
        ]
        for c in copies:
            c.wait()

        def select(k, _):
            p0 = k * _L
            pv = lax.iota(jnp.int32, _L) + p0
            hv = hoff_v[pl.ds(p0, _L)]
            for c in range(D):
                vals = plsc.load_gather(rows_v, [pv, hv + c])
                outt_v[c, pl.ds(p0, _L)] = vals
            return 0

        lax.fori_loop(0, b_per_w // _L, select, 0)
        pltpu.sync_copy(outt_v, out_hbm.at[:, pl.ds(base, b_per_w)])

    return gather_kernel


def kernel(labels, table, num_objs):
    B = labels.shape[0]
    D = table.shape[1]
    n = math.isqrt(table.shape[0])
    l0 = labels[:, 0]
    l1 = labels[:, 1]
    tab_rs = table.reshape(table.shape[0] // 2, 2 * D)
    out_t = _make_gather(B, D, n)(l0, l1, tab_rs)
    return out_t.T
